# baseline (speedup 1.0000x reference)
import functools

import jax
import jax.numpy as jnp
from jax import lax
from jax.experimental import pallas as pl
from jax.experimental.pallas import tpu as pltpu
from jax.experimental.pallas import tpu_sc as plsc

_EMB = 1024
_NC = 2
_NS = 16
_NW = _NC * _NS
_CH = 16
_NBUF = 4


@functools.partial(jax.jit, static_argnums=())
def _gather_rows(idx, table):
    nb, seq = idx.shape
    B = nb * seq
    b_per_w = B // _NW
    w_per_b = seq // b_per_w
    nch = b_per_w // _CH
    mesh = plsc.VectorSubcoreMesh(core_axis_name="c", subcore_axis_name="s")

    @functools.partial(
        pl.kernel,
        out_type=jax.ShapeDtypeStruct((nb, seq, _EMB), jnp.float32),
        mesh=mesh,
        scratch_types=[
            pltpu.VMEM((b_per_w,), jnp.int32),
            pltpu.VMEM((_NBUF, _CH, _EMB), jnp.float32),
            [pltpu.SemaphoreType.DMA] * _NBUF,
            [pltpu.SemaphoreType.DMA] * _NBUF,
        ],
    )
    def k(idx_hbm, table_hbm, out_hbm, idx_v, bufs, gsems, osems):
        wid = lax.axis_index("s") * _NC + lax.axis_index("c")
        bi = wid // w_per_b
        base = (wid % w_per_b) * b_per_w
        pltpu.sync_copy(idx_hbm.at[bi].at[pl.ds(base, b_per_w)], idx_v)

        def start_gather(c, b):
            off = pl.multiple_of((c % (8192 // _CH)) * _CH, _CH)
            pltpu.async_copy(table_hbm.at[pl.ds(off, _CH)], bufs.at[b], gsems[b])

        def wait_gather(b):
            pltpu.make_async_copy(
                table_hbm.at[pl.ds(0, _CH)], bufs.at[b], gsems[b]
            ).wait()

        def start_write(c, b):
            off = pl.multiple_of(c * _CH, _CH)
            pltpu.async_copy(
                bufs.at[b], out_hbm.at[bi].at[pl.ds(base + off, _CH)], osems[b]
            )

        def wait_write(b):
            pltpu.make_async_copy(
                bufs.at[b], out_hbm.at[bi].at[pl.ds(base, _CH)], osems[b]
            ).wait()

        for b in range(_NBUF):
            start_gather(b, b)

        @pl.loop(0, nch // _NBUF)
        def _visits(g):
            for b in range(_NBUF):
                c = g * _NBUF + b
                wait_gather(b)
                start_write(c, b)
                bp = (b - 1) % _NBUF
                cn = c - 1 + _NBUF

                @pl.when((c >= 1) & (cn < nch))
                def _():
                    wait_write(bp)
                    start_gather(cn, bp)

        for b in range(_NBUF):
            wait_write(b)

    return k(idx, table)


def kernel(input_text, encodings):
    return _gather_rows(input_text.astype(jnp.int32), encodings)


# speedup vs baseline: 1.3931x; 1.3931x over previous
import functools

import jax
import jax.numpy as jnp
from jax import lax
from jax.experimental import pallas as pl
from jax.experimental.pallas import tpu as pltpu
from jax.experimental.pallas import tpu_sc as plsc

_EMB = 1024
_NC = 2
_NS = 16
_NW = _NC * _NS
_CH = 16
_NBUF = 4


@functools.partial(jax.jit, static_argnums=())
def _gather_rows(idx, table):
    nb, seq = idx.shape
    B = nb * seq
    b_per_w = B // _NW
    w_per_b = seq // b_per_w
    nch = b_per_w // _CH
    mesh = plsc.VectorSubcoreMesh(core_axis_name="c", subcore_axis_name="s")

    @functools.partial(
        pl.kernel,
        out_type=jax.ShapeDtypeStruct((nb, seq, _EMB), jnp.float32),
        mesh=mesh,
        scratch_types=[
            pltpu.VMEM((b_per_w,), jnp.int32),
            pltpu.VMEM((_NBUF, _CH, _EMB), jnp.float32),
            [pltpu.SemaphoreType.DMA] * _NBUF,
            [pltpu.SemaphoreType.DMA] * _NBUF,
        ],
    )
    def k(idx_hbm, table_hbm, out_hbm, idx_v, bufs, gsems, osems):
        wid = lax.axis_index("s") * _NC + lax.axis_index("c")
        bi = wid // w_per_b
        base = (wid % w_per_b) * b_per_w
        pltpu.sync_copy(idx_hbm.at[bi].at[pl.ds(base, b_per_w)], idx_v)

        def start_gather(c, b):
            off = pl.multiple_of(c * _CH, _CH)
            pltpu.async_copy(
                table_hbm.at[idx_v.at[pl.ds(off, _CH)]], bufs.at[b], gsems[b]
            )

        def wait_gather(b):
            pltpu.make_async_copy(
                table_hbm.at[idx_v.at[pl.ds(0, _CH)]], bufs.at[b], gsems[b]
            ).wait()

        def start_write(c, b):
            off = pl.multiple_of(c * _CH, _CH)
            pltpu.async_copy(
                bufs.at[b], out_hbm.at[bi].at[pl.ds(base + off, _CH)], osems[b]
            )

        def wait_write(b):
            pltpu.make_async_copy(
                bufs.at[b], out_hbm.at[bi].at[pl.ds(base, _CH)], osems[b]
            ).wait()

        for b in range(_NBUF):
            start_gather(b, b)

        @pl.loop(0, nch // _NBUF)
        def _visits(g):
            for b in range(_NBUF):
                c = g * _NBUF + b
                wait_gather(b)
                start_write(c, b)
                bp = (b - 1) % _NBUF
                cn = c - 1 + _NBUF

                @pl.when((c >= 1) & (cn < nch))
                def _():
                    wait_write(bp)
                    start_gather(cn, bp)

        for b in range(_NBUF):
            wait_write(b)

    return k(idx, table)


def kernel(input_text, encodings):
    return _gather_rows(input_text.astype(jnp.int32), encodings)
